# baseline (speedup 1.0000x reference)
import functools

import jax
import jax.numpy as jnp
from jax import lax
from jax.experimental import pallas as pl
from jax.experimental.pallas import tpu as pltpu
from jax.experimental.pallas import tpu_sc as plsc

_N = 10000
_G = 64
_NC = 2
_NS = 16
_NW = _NC * _NS
_CHUNK = 128
_N_PAD = 10240
_ROWS_PER_TILE = _N_PAD // _NS
_DEG_W = 16
_DEG_SC = 128

_mesh = plsc.VectorSubcoreMesh(core_axis_name="c", subcore_axis_name="s")


def _make_deg(n_chunks):
    @functools.partial(
        pl.kernel,
        mesh=_mesh,
        out_type=jax.ShapeDtypeStruct((_NC, _N_PAD, _DEG_SC), jnp.float32),
        scratch_types=[
            pltpu.VMEM((n_chunks, _CHUNK), jnp.int32),
            pltpu.VMEM((_CHUNK, _DEG_SC), jnp.float32),
            pltpu.VMEM_SHARED((_N_PAD, _DEG_SC), jnp.float32),
        ],
    )
    def k(dst_hbm, ones_hbm, zero_hbm, out_hbm, dst_v, ones_v, acc):
        c = lax.axis_index("c")
        s = lax.axis_index("s")
        wid = c * _NS + s
        pltpu.sync_copy(dst_hbm.at[wid], dst_v)
        pltpu.sync_copy(ones_hbm, ones_v)
        base = s * _ROWS_PER_TILE

        @pl.loop(0, _ROWS_PER_TILE // _CHUNK)
        def _(i):
            pltpu.sync_copy(zero_hbm, acc.at[pl.ds(base + i * _CHUNK, _CHUNK)])

        plsc.subcore_barrier()

        @pl.loop(0, n_chunks)
        def _(j):
            pltpu.sync_copy(ones_v, acc.at[dst_v.at[j]], add=True)

        plsc.subcore_barrier()
        pltpu.sync_copy(
            acc.at[pl.ds(base, _ROWS_PER_TILE)],
            out_hbm.at[c, pl.ds(base, _ROWS_PER_TILE)],
        )

    return k


def _make_scatter(n_chunks, f):
    assert n_chunks % 4 == 0
    half = n_chunks // 2

    @functools.partial(
        pl.kernel,
        mesh=_mesh,
        out_type=jax.ShapeDtypeStruct((_NC, _N_PAD, f), jnp.float32),
        scratch_types=[
            pltpu.VMEM((half, _CHUNK), jnp.int32),
            pltpu.VMEM((half, _CHUNK), jnp.int32),
            pltpu.VMEM((_CHUNK, f), jnp.float32),
            pltpu.VMEM((_CHUNK, f), jnp.float32),
            pltpu.VMEM_SHARED((_N_PAD, f), jnp.float32),
            pltpu.SemaphoreType.DMA,
            pltpu.SemaphoreType.DMA,
        ],
    )
    def k(h_hbm, src_hbm, dst_hbm, zero_hbm, out_hbm,
          src_v, dst_v, rows0, rows1, acc, sem0, sem1):
        c = lax.axis_index("c")
        s = lax.axis_index("s")
        wid = c * _NS + s
        base = s * _ROWS_PER_TILE

        @pl.loop(0, _ROWS_PER_TILE // _CHUNK)
        def _(i):
            pltpu.sync_copy(zero_hbm, acc.at[pl.ds(base + i * _CHUNK, _CHUNK)])

        plsc.subcore_barrier()

        for h0 in (0, half):
            pltpu.sync_copy(src_hbm.at[wid, pl.ds(h0, half)], src_v)
            pltpu.sync_copy(dst_hbm.at[wid, pl.ds(h0, half)], dst_v)

            @pl.loop(0, half, step=2)
            def _(j):
                pltpu.sync_copy(h_hbm.at[src_v.at[j]], rows0)
                pltpu.sync_copy(rows0, acc.at[dst_v.at[j]], add=True)
                pltpu.sync_copy(h_hbm.at[src_v.at[j + 1]], rows1)
                pltpu.sync_copy(rows1, acc.at[dst_v.at[j + 1]], add=True)

        plsc.subcore_barrier()
        pltpu.sync_copy(
            acc.at[pl.ds(base, _ROWS_PER_TILE)],
            out_hbm.at[c, pl.ds(base, _ROWS_PER_TILE)],
        )

    return k


_B = 1000


def _dinv(d0, d1):
    return 1.0 / jnp.sqrt(1.0 + d0[:, :1] + d1[:, :1])


def _mm_first_body(x_ref, d0_ref, d1_ref, w_ref, o_ref):
    dinv = _dinv(d0_ref[...], d1_ref[...])
    o_ref[...] = dinv * jnp.dot(
        x_ref[...], w_ref[...], preferred_element_type=jnp.float32
    )


def _mm_first(x, d0, d1, w):
    din, dout = w.shape
    return pl.pallas_call(
        _mm_first_body,
        grid=(_N // _B,),
        in_specs=[
            pl.BlockSpec((_B, din), lambda i: (i, 0)),
            pl.BlockSpec((_B, _DEG_W), lambda i: (i, 0)),
            pl.BlockSpec((_B, _DEG_W), lambda i: (i, 0)),
            pl.BlockSpec((din, dout), lambda i: (0, 0)),
        ],
        out_specs=pl.BlockSpec((_B, dout), lambda i: (i, 0)),
        out_shape=jax.ShapeDtypeStruct((_N, dout), jnp.float32),
    )(x, d0, d1, w)


def _layer_body(p0_ref, p1_ref, h_ref, d0_ref, d1_ref, b_ref, w_ref, o_ref):
    dinv = _dinv(d0_ref[...], d1_ref[...])
    xb = jnp.maximum(
        dinv * (p0_ref[...] + p1_ref[...] + h_ref[...]) + b_ref[...], 0.0
    )
    o_ref[...] = dinv * jnp.dot(
        xb, w_ref[...], preferred_element_type=jnp.float32
    )


def _layer(p0, p1, h, d0, d1, b, w):
    din, dout = w.shape
    return pl.pallas_call(
        _layer_body,
        grid=(_N // _B,),
        in_specs=[
            pl.BlockSpec((_B, din), lambda i: (i, 0)),
            pl.BlockSpec((_B, din), lambda i: (i, 0)),
            pl.BlockSpec((_B, din), lambda i: (i, 0)),
            pl.BlockSpec((_B, _DEG_W), lambda i: (i, 0)),
            pl.BlockSpec((_B, _DEG_W), lambda i: (i, 0)),
            pl.BlockSpec((1, din), lambda i: (0, 0)),
            pl.BlockSpec((din, dout), lambda i: (0, 0)),
        ],
        out_specs=pl.BlockSpec((_B, dout), lambda i: (i, 0)),
        out_shape=jax.ShapeDtypeStruct((_N, dout), jnp.float32),
    )(p0, p1, h, d0, d1, b, w)


def _final_body(
    p0_ref, p1_ref, h_ref, d0_ref, d1_ref, b_ref, bt_ref, wl_ref, bl_ref,
    o_ref, sums, cnts
):
    i = pl.program_id(0)
    dinv = _dinv(d0_ref[...], d1_ref[...])
    xb = jnp.maximum(
        dinv * (p0_ref[...] + p1_ref[...] + h_ref[...]) + b_ref[...], 0.0
    )
    segs = lax.broadcasted_iota(jnp.int32, (_B, _G), 1).astype(jnp.float32)
    mask = (bt_ref[...][:, :1] == segs).astype(jnp.float32)
    dims = (((0,), (0,)), ((), ()))
    psum = lax.dot_general(mask, xb, dims, preferred_element_type=jnp.float32, precision=lax.Precision.HIGHEST)
    pcnt = lax.dot_general(
        mask, jnp.ones_like(xb), dims, preferred_element_type=jnp.float32, precision=lax.Precision.HIGHEST
    )

    @pl.when(i == 0)
    def _():
        sums[...] = jnp.zeros_like(sums)
        cnts[...] = jnp.zeros_like(cnts)

    sums[...] += psum
    cnts[...] += pcnt

    @pl.when(i == (_N // _B) - 1)
    def _():
        pooled = sums[...] / jnp.maximum(cnts[...], 1.0)
        mu = jnp.mean(pooled, axis=1, keepdims=True)
        var = jnp.mean((pooled - mu) ** 2, axis=1, keepdims=True)
        xn = (pooled - mu) / jnp.sqrt(var + 1e-5)
        o_ref[...] = (
            jnp.dot(xn, wl_ref[...], preferred_element_type=jnp.float32)
            + bl_ref[...]
        )


def _final(p0, p1, h, d0, d1, b, batchf, wl, bl2):
    din = h.shape[1]
    return pl.pallas_call(
        _final_body,
        grid=(_N // _B,),
        in_specs=[
            pl.BlockSpec((_B, din), lambda i: (i, 0)),
            pl.BlockSpec((_B, din), lambda i: (i, 0)),
            pl.BlockSpec((_B, din), lambda i: (i, 0)),
            pl.BlockSpec((_B, _DEG_W), lambda i: (i, 0)),
            pl.BlockSpec((_B, _DEG_W), lambda i: (i, 0)),
            pl.BlockSpec((1, din), lambda i: (0, 0)),
            pl.BlockSpec((_B, _DEG_W), lambda i: (i, 0)),
            pl.BlockSpec((din, 1), lambda i: (0, 0)),
            pl.BlockSpec((1, 1), lambda i: (0, 0)),
        ],
        out_specs=pl.BlockSpec((_G, 1), lambda i: (0, 0)),
        out_shape=jax.ShapeDtypeStruct((_G, 1), jnp.float32),
        scratch_shapes=[
            pltpu.VMEM((_G, din), jnp.float32),
            pltpu.VMEM((_G, din), jnp.float32),
        ],
    )(p0, p1, h, d0, d1, b, batchf, wl, bl2)


def kernel(x, edge_index, batch, W1, b1, W2, b2, W3, b3, W4, b4, W5, b5, Wl, bl):
    e = edge_index.shape[1]
    n_chunks = -(-e // (_NW * _CHUNK))
    n_chunks = -(-n_chunks // 4) * 4
    e_pad = n_chunks * _NW * _CHUNK
    pad = e_pad - e
    src = jnp.concatenate(
        [edge_index[0], jnp.zeros((pad,), jnp.int32)]
    ).reshape(_NW, n_chunks, _CHUNK)
    dst = jnp.concatenate(
        [edge_index[1], jnp.full((pad,), _N, jnp.int32)]
    ).reshape(_NW, n_chunks, _CHUNK)

    zeros128 = jnp.zeros((_CHUNK, 128), jnp.float32)
    onesd = jnp.ones((_CHUNK, _DEG_SC), jnp.float32)

    dpart = _make_deg(n_chunks)(dst, onesd, zeros128)
    d0 = dpart[0, :_N, :_DEG_W]
    d1 = dpart[1, :_N, :_DEG_W]

    def pad_w(w):
        return jnp.zeros((128, 128), jnp.float32).at[: w.shape[0], : w.shape[1]].set(w)

    def pad_b(b):
        return jnp.zeros((128,), jnp.float32).at[: b.shape[0]].set(b)

    scatter = _make_scatter(n_chunks, 128)
    h = _mm_first(x, d0, d1, W1)
    weights = [(b1, pad_w(W2)), (pad_b(b2), pad_w(W3)),
               (pad_b(b3), pad_w(W4)), (pad_b(b4), pad_w(W5))]
    for b_prev, w_next in weights:
        p = scatter(h, src, dst, zeros128)
        h = _layer(p[0, :_N], p[1, :_N], h, d0, d1, b_prev[None, :], w_next)

    p = scatter(h, src, dst, zeros128)
    batchf = jnp.broadcast_to(
        batch.astype(jnp.float32)[:, None], (_N, _DEG_W)
    )
    return _final(
        p[0, :_N], p[1, :_N], h, d0, d1, b5[None, :], batchf, Wl,
        bl.reshape(1, 1)
    )


# speedup vs baseline: 2.6614x; 2.6614x over previous
import functools

import jax
import jax.numpy as jnp
from jax import lax
from jax.experimental import pallas as pl
from jax.experimental.pallas import tpu as pltpu
from jax.experimental.pallas import tpu_sc as plsc

_N = 10000
_G = 64
_NC = 2
_NS = 16
_NW = _NC * _NS
_CHUNK = 128
_N_PAD = 10240
_ROWS_PER_TILE = _N_PAD // _NS
_DEG_W = 16
_DEG_SC = 128

_mesh = plsc.VectorSubcoreMesh(core_axis_name="c", subcore_axis_name="s")


def _make_deg(n_chunks):
    @functools.partial(
        pl.kernel,
        mesh=_mesh,
        out_type=jax.ShapeDtypeStruct((_NC, _N_PAD, _DEG_SC), jnp.float32),
        scratch_types=[
            pltpu.VMEM((n_chunks, _CHUNK), jnp.int32),
            pltpu.VMEM((_CHUNK, _DEG_SC), jnp.float32),
            pltpu.VMEM_SHARED((_N_PAD, _DEG_SC), jnp.float32),
        ],
    )
    def k(dst_hbm, ones_hbm, zero_hbm, out_hbm, dst_v, ones_v, acc):
        c = lax.axis_index("c")
        s = lax.axis_index("s")
        wid = c * _NS + s
        pltpu.sync_copy(dst_hbm.at[wid], dst_v)
        pltpu.sync_copy(ones_hbm, ones_v)
        base = s * _ROWS_PER_TILE

        @pl.loop(0, _ROWS_PER_TILE // _CHUNK)
        def _(i):
            pltpu.sync_copy(zero_hbm, acc.at[pl.ds(base + i * _CHUNK, _CHUNK)])

        plsc.subcore_barrier()

        @pl.loop(0, n_chunks)
        def _(j):
            pltpu.sync_copy(ones_v, acc.at[dst_v.at[j]], add=True)

        plsc.subcore_barrier()
        pltpu.sync_copy(
            acc.at[pl.ds(base, _ROWS_PER_TILE)],
            out_hbm.at[c, pl.ds(base, _ROWS_PER_TILE)],
        )

    return k


def _make_scatter(n_chunks, f):
    assert n_chunks % 4 == 0
    half = n_chunks // 2

    @functools.partial(
        pl.kernel,
        mesh=_mesh,
        out_type=jax.ShapeDtypeStruct((_NC, _N_PAD, f), jnp.float32),
        scratch_types=[
            pltpu.VMEM((half, _CHUNK), jnp.int32),
            pltpu.VMEM((half, _CHUNK), jnp.int32),
            pltpu.VMEM((_CHUNK, f), jnp.float32),
            pltpu.VMEM((_CHUNK, f), jnp.float32),
            pltpu.VMEM_SHARED((_N_PAD, f), jnp.float32),
            pltpu.SemaphoreType.DMA,
            pltpu.SemaphoreType.DMA,
        ],
    )
    def k(h_hbm, src_hbm, dst_hbm, zero_hbm, out_hbm,
          src_v, dst_v, rows0, rows1, acc, sem0, sem1):
        c = lax.axis_index("c")
        s = lax.axis_index("s")
        wid = c * _NS + s
        base = s * _ROWS_PER_TILE

        @pl.loop(0, _ROWS_PER_TILE // _CHUNK)
        def _(i):
            pltpu.sync_copy(zero_hbm, acc.at[pl.ds(base + i * _CHUNK, _CHUNK)])

        plsc.subcore_barrier()

        for h0 in (0, half):
            pltpu.sync_copy(src_hbm.at[wid, pl.ds(h0, half)], src_v)
            pltpu.sync_copy(dst_hbm.at[wid, pl.ds(h0, half)], dst_v)

            @pl.loop(0, half, step=2)
            def _(j):
                pltpu.sync_copy(h_hbm.at[src_v.at[j]], rows0)
                pltpu.sync_copy(rows0, acc.at[dst_v.at[j]], add=True)
                pltpu.sync_copy(h_hbm.at[src_v.at[j + 1]], rows1)
                pltpu.sync_copy(rows1, acc.at[dst_v.at[j + 1]], add=True)

        plsc.subcore_barrier()
        pltpu.sync_copy(
            acc.at[pl.ds(base, _ROWS_PER_TILE)],
            out_hbm.at[c, pl.ds(base, _ROWS_PER_TILE)],
        )

    return k


_B = 1000


def _dinv(d0, d1):
    return 1.0 / jnp.sqrt(1.0 + d0[:, :1] + d1[:, :1])


def _mm_first_body(x_ref, d0_ref, d1_ref, w_ref, o_ref):
    dinv = _dinv(d0_ref[...], d1_ref[...])
    o_ref[...] = dinv * jnp.dot(
        x_ref[...], w_ref[...], preferred_element_type=jnp.float32
    )


def _mm_first(x, d0, d1, w):
    din, dout = w.shape
    return pl.pallas_call(
        _mm_first_body,
        grid=(_N // _B,),
        in_specs=[
            pl.BlockSpec((_B, din), lambda i: (i, 0)),
            pl.BlockSpec((_B, _DEG_W), lambda i: (i, 0)),
            pl.BlockSpec((_B, _DEG_W), lambda i: (i, 0)),
            pl.BlockSpec((din, dout), lambda i: (0, 0)),
        ],
        out_specs=pl.BlockSpec((_B, dout), lambda i: (i, 0)),
        out_shape=jax.ShapeDtypeStruct((_N, dout), jnp.float32),
    )(x, d0, d1, w)


def _layer_body(p0_ref, p1_ref, h_ref, d0_ref, d1_ref, b_ref, w_ref, o_ref):
    dinv = _dinv(d0_ref[...], d1_ref[...])
    xb = jnp.maximum(
        dinv * (p0_ref[...] + p1_ref[...] + h_ref[...]) + b_ref[...], 0.0
    )
    o_ref[...] = dinv * jnp.dot(
        xb, w_ref[...], preferred_element_type=jnp.float32
    )


def _layer(p0, p1, h, d0, d1, b, w):
    din, dout = w.shape
    return pl.pallas_call(
        _layer_body,
        grid=(_N // _B,),
        in_specs=[
            pl.BlockSpec((_B, din), lambda i: (i, 0)),
            pl.BlockSpec((_B, din), lambda i: (i, 0)),
            pl.BlockSpec((_B, din), lambda i: (i, 0)),
            pl.BlockSpec((_B, _DEG_W), lambda i: (i, 0)),
            pl.BlockSpec((_B, _DEG_W), lambda i: (i, 0)),
            pl.BlockSpec((1, din), lambda i: (0, 0)),
            pl.BlockSpec((din, dout), lambda i: (0, 0)),
        ],
        out_specs=pl.BlockSpec((_B, dout), lambda i: (i, 0)),
        out_shape=jax.ShapeDtypeStruct((_N, dout), jnp.float32),
    )(p0, p1, h, d0, d1, b, w)


def _final_body(
    p0_ref, p1_ref, h_ref, d0_ref, d1_ref, b_ref, bt_ref, wl_ref, bl_ref,
    o_ref, sums, cnts
):
    i = pl.program_id(0)
    dinv = _dinv(d0_ref[...], d1_ref[...])
    xb = jnp.maximum(
        dinv * (p0_ref[...] + p1_ref[...] + h_ref[...]) + b_ref[...], 0.0
    )
    segs = lax.broadcasted_iota(jnp.int32, (_B, _G), 1).astype(jnp.float32)
    mask = (bt_ref[...][:, :1] == segs).astype(jnp.float32)
    dims = (((0,), (0,)), ((), ()))
    psum = lax.dot_general(mask, xb, dims, preferred_element_type=jnp.float32, precision=lax.Precision.HIGHEST)
    pcnt = lax.dot_general(
        mask, jnp.ones_like(xb), dims, preferred_element_type=jnp.float32, precision=lax.Precision.HIGHEST
    )

    @pl.when(i == 0)
    def _():
        sums[...] = jnp.zeros_like(sums)
        cnts[...] = jnp.zeros_like(cnts)

    sums[...] += psum
    cnts[...] += pcnt

    @pl.when(i == (_N // _B) - 1)
    def _():
        pooled = sums[...] / jnp.maximum(cnts[...], 1.0)
        mu = jnp.mean(pooled, axis=1, keepdims=True)
        var = jnp.mean((pooled - mu) ** 2, axis=1, keepdims=True)
        xn = (pooled - mu) / jnp.sqrt(var + 1e-5)
        o_ref[...] = (
            jnp.dot(xn, wl_ref[...], preferred_element_type=jnp.float32)
            + bl_ref[...]
        )


def _final(p0, p1, h, d0, d1, b, batchf, wl, bl2):
    din = h.shape[1]
    return pl.pallas_call(
        _final_body,
        grid=(_N // _B,),
        in_specs=[
            pl.BlockSpec((_B, din), lambda i: (i, 0)),
            pl.BlockSpec((_B, din), lambda i: (i, 0)),
            pl.BlockSpec((_B, din), lambda i: (i, 0)),
            pl.BlockSpec((_B, _DEG_W), lambda i: (i, 0)),
            pl.BlockSpec((_B, _DEG_W), lambda i: (i, 0)),
            pl.BlockSpec((1, din), lambda i: (0, 0)),
            pl.BlockSpec((_B, _DEG_W), lambda i: (i, 0)),
            pl.BlockSpec((din, 1), lambda i: (0, 0)),
            pl.BlockSpec((1, 1), lambda i: (0, 0)),
        ],
        out_specs=pl.BlockSpec((_G, 1), lambda i: (0, 0)),
        out_shape=jax.ShapeDtypeStruct((_G, 1), jnp.float32),
        scratch_shapes=[
            pltpu.VMEM((_G, din), jnp.float32),
            pltpu.VMEM((_G, din), jnp.float32),
        ],
    )(p0, p1, h, d0, d1, b, batchf, wl, bl2)


def kernel(x, edge_index, batch, W1, b1, W2, b2, W3, b3, W4, b4, W5, b5, Wl, bl):
    e = edge_index.shape[1]
    n_chunks = -(-e // (_NW * _CHUNK))
    n_chunks = -(-n_chunks // 4) * 4
    e_pad = n_chunks * _NW * _CHUNK
    pad = e_pad - e
    pad_dst = _N + (jnp.arange(pad, dtype=jnp.int32) % (_N_PAD - _N))
    pad_src = jnp.arange(pad, dtype=jnp.int32) % _N
    src = jnp.concatenate([edge_index[0], pad_src]).reshape(_NW, n_chunks, _CHUNK)
    dst = jnp.concatenate([edge_index[1], pad_dst]).reshape(_NW, n_chunks, _CHUNK)

    zeros128 = jnp.zeros((_CHUNK, 128), jnp.float32)
    onesd = jnp.ones((_CHUNK, _DEG_SC), jnp.float32)

    dpart = _make_deg(n_chunks)(dst, onesd, zeros128)
    d0 = dpart[0, :_N, :_DEG_W]
    d1 = dpart[1, :_N, :_DEG_W]

    def pad_w(w):
        return jnp.zeros((128, 128), jnp.float32).at[: w.shape[0], : w.shape[1]].set(w)

    def pad_b(b):
        return jnp.zeros((128,), jnp.float32).at[: b.shape[0]].set(b)

    scatter = _make_scatter(n_chunks, 128)
    h = _mm_first(x, d0, d1, W1)
    weights = [(b1, pad_w(W2)), (pad_b(b2), pad_w(W3)),
               (pad_b(b3), pad_w(W4)), (pad_b(b4), pad_w(W5))]
    for b_prev, w_next in weights:
        p = scatter(h, src, dst, zeros128)
        h = _layer(p[0, :_N], p[1, :_N], h, d0, d1, b_prev[None, :], w_next)

    p = scatter(h, src, dst, zeros128)
    batchf = jnp.broadcast_to(
        batch.astype(jnp.float32)[:, None], (_N, _DEG_W)
    )
    return _final(
        p[0, :_N], p[1, :_N], h, d0, d1, b5[None, :], batchf, Wl,
        bl.reshape(1, 1)
    )


# speedup vs baseline: 3.6895x; 1.3863x over previous
import functools

import jax
import jax.numpy as jnp
from jax import lax
from jax.experimental import pallas as pl
from jax.experimental.pallas import tpu as pltpu
from jax.experimental.pallas import tpu_sc as plsc

_N = 10000
_G = 64
_NC = 2
_NS = 16
_NW = _NC * _NS
_CHUNK = 128
_N_PAD = 10240
_ROWS_PER_TILE = _N_PAD // _NS
_DEG_W = 16
_DEG_SC = 128

_mesh = plsc.VectorSubcoreMesh(core_axis_name="c", subcore_axis_name="s")


def _make_deg(n_chunks):
    @functools.partial(
        pl.kernel,
        mesh=_mesh,
        out_type=jax.ShapeDtypeStruct((_NC, _N_PAD, _DEG_SC), jnp.float32),
        scratch_types=[
            pltpu.VMEM((n_chunks, _CHUNK), jnp.int32),
            pltpu.VMEM((_CHUNK, _DEG_SC), jnp.float32),
            pltpu.VMEM_SHARED((_N_PAD, _DEG_SC), jnp.float32),
        ],
    )
    def k(dst_hbm, ones_hbm, zero_hbm, out_hbm, dst_v, ones_v, acc):
        c = lax.axis_index("c")
        s = lax.axis_index("s")
        wid = c * _NS + s
        pltpu.sync_copy(dst_hbm.at[wid], dst_v)
        pltpu.sync_copy(ones_hbm, ones_v)
        base = s * _ROWS_PER_TILE

        @pl.loop(0, _ROWS_PER_TILE // _CHUNK)
        def _(i):
            pltpu.sync_copy(zero_hbm, acc.at[pl.ds(base + i * _CHUNK, _CHUNK)])

        plsc.subcore_barrier()

        @pl.loop(0, n_chunks)
        def _(j):
            pltpu.sync_copy(ones_v, acc.at[dst_v.at[j]], add=True)

        plsc.subcore_barrier()
        pltpu.sync_copy(
            acc.at[pl.ds(base, _ROWS_PER_TILE)],
            out_hbm.at[c, pl.ds(base, _ROWS_PER_TILE)],
        )

    return k


def _make_scatter(n_chunks, f):
    assert n_chunks % 4 == 0
    half = n_chunks // 2

    @functools.partial(
        pl.kernel,
        mesh=_mesh,
        out_type=jax.ShapeDtypeStruct((_NC, _N_PAD, f), jnp.float32),
        scratch_types=[
            pltpu.VMEM((half, _CHUNK), jnp.int32),
            pltpu.VMEM((half, _CHUNK), jnp.int32),
            pltpu.VMEM((_CHUNK, f), jnp.float32),
            pltpu.VMEM((_CHUNK, f), jnp.float32),
            pltpu.VMEM_SHARED((_N_PAD, f), jnp.float32),
            pltpu.SemaphoreType.DMA,
            pltpu.SemaphoreType.DMA,
        ],
    )
    def k(h_hbm, src_hbm, dst_hbm, zero_hbm, out_hbm,
          src_v, dst_v, rows0, rows1, acc, sem0, sem1):
        c = lax.axis_index("c")
        s = lax.axis_index("s")
        wid = c * _NS + s
        base = s * _ROWS_PER_TILE

        @pl.loop(0, _ROWS_PER_TILE // _CHUNK)
        def _(i):
            pltpu.sync_copy(zero_hbm, acc.at[pl.ds(base + i * _CHUNK, _CHUNK)])

        plsc.subcore_barrier()

        for h0 in (0, half):
            pltpu.sync_copy(src_hbm.at[wid, pl.ds(h0, half)], src_v)
            pltpu.sync_copy(dst_hbm.at[wid, pl.ds(h0, half)], dst_v)

            pltpu.async_copy(h_hbm.at[src_v.at[0]], rows0, sem0)

            @pl.loop(0, half - 2, step=2)
            def _(j):
                pltpu.async_copy(h_hbm.at[src_v.at[j + 1]], rows1, sem1)
                pltpu.make_async_copy(h_hbm.at[src_v.at[j]], rows0, sem0).wait()
                pltpu.sync_copy(rows0, acc.at[dst_v.at[j]], add=True)
                pltpu.async_copy(h_hbm.at[src_v.at[j + 2]], rows0, sem0)
                pltpu.make_async_copy(h_hbm.at[src_v.at[j + 1]], rows1, sem1).wait()
                pltpu.sync_copy(rows1, acc.at[dst_v.at[j + 1]], add=True)

            jt = half - 2
            pltpu.async_copy(h_hbm.at[src_v.at[jt + 1]], rows1, sem1)
            pltpu.make_async_copy(h_hbm.at[src_v.at[jt]], rows0, sem0).wait()
            pltpu.sync_copy(rows0, acc.at[dst_v.at[jt]], add=True)
            pltpu.make_async_copy(h_hbm.at[src_v.at[jt + 1]], rows1, sem1).wait()
            pltpu.sync_copy(rows1, acc.at[dst_v.at[jt + 1]], add=True)

        plsc.subcore_barrier()
        pltpu.sync_copy(
            acc.at[pl.ds(base, _ROWS_PER_TILE)],
            out_hbm.at[c, pl.ds(base, _ROWS_PER_TILE)],
        )

    return k


_B = 1000


def _dinv(d0, d1):
    return 1.0 / jnp.sqrt(1.0 + d0[:, :1] + d1[:, :1])


def _mm_first_body(x_ref, d0_ref, d1_ref, w_ref, o_ref):
    dinv = _dinv(d0_ref[...], d1_ref[...])
    o_ref[...] = dinv * jnp.dot(
        x_ref[...], w_ref[...], preferred_element_type=jnp.float32
    )


def _mm_first(x, d0, d1, w):
    din, dout = w.shape
    return pl.pallas_call(
        _mm_first_body,
        grid=(_N // _B,),
        in_specs=[
            pl.BlockSpec((_B, din), lambda i: (i, 0)),
            pl.BlockSpec((_B, _DEG_W), lambda i: (i, 0)),
            pl.BlockSpec((_B, _DEG_W), lambda i: (i, 0)),
            pl.BlockSpec((din, dout), lambda i: (0, 0)),
        ],
        out_specs=pl.BlockSpec((_B, dout), lambda i: (i, 0)),
        out_shape=jax.ShapeDtypeStruct((_N, dout), jnp.float32),
    )(x, d0, d1, w)


def _layer_body(p0_ref, p1_ref, h_ref, d0_ref, d1_ref, b_ref, w_ref, o_ref):
    dinv = _dinv(d0_ref[...], d1_ref[...])
    xb = jnp.maximum(
        dinv * (p0_ref[...] + p1_ref[...] + h_ref[...]) + b_ref[...], 0.0
    )
    o_ref[...] = dinv * jnp.dot(
        xb, w_ref[...], preferred_element_type=jnp.float32
    )


def _layer(p0, p1, h, d0, d1, b, w):
    din, dout = w.shape
    return pl.pallas_call(
        _layer_body,
        grid=(_N // _B,),
        in_specs=[
            pl.BlockSpec((_B, din), lambda i: (i, 0)),
            pl.BlockSpec((_B, din), lambda i: (i, 0)),
            pl.BlockSpec((_B, din), lambda i: (i, 0)),
            pl.BlockSpec((_B, _DEG_W), lambda i: (i, 0)),
            pl.BlockSpec((_B, _DEG_W), lambda i: (i, 0)),
            pl.BlockSpec((1, din), lambda i: (0, 0)),
            pl.BlockSpec((din, dout), lambda i: (0, 0)),
        ],
        out_specs=pl.BlockSpec((_B, dout), lambda i: (i, 0)),
        out_shape=jax.ShapeDtypeStruct((_N, dout), jnp.float32),
    )(p0, p1, h, d0, d1, b, w)


def _final_body(
    p0_ref, p1_ref, h_ref, d0_ref, d1_ref, b_ref, bt_ref, wl_ref, bl_ref,
    o_ref, sums, cnts
):
    i = pl.program_id(0)
    dinv = _dinv(d0_ref[...], d1_ref[...])
    xb = jnp.maximum(
        dinv * (p0_ref[...] + p1_ref[...] + h_ref[...]) + b_ref[...], 0.0
    )
    segs = lax.broadcasted_iota(jnp.int32, (_B, _G), 1).astype(jnp.float32)
    mask = (bt_ref[...][:, :1] == segs).astype(jnp.float32)
    dims = (((0,), (0,)), ((), ()))
    psum = lax.dot_general(mask, xb, dims, preferred_element_type=jnp.float32, precision=lax.Precision.HIGHEST)
    pcnt = lax.dot_general(
        mask, jnp.ones_like(xb), dims, preferred_element_type=jnp.float32, precision=lax.Precision.HIGHEST
    )

    @pl.when(i == 0)
    def _():
        sums[...] = jnp.zeros_like(sums)
        cnts[...] = jnp.zeros_like(cnts)

    sums[...] += psum
    cnts[...] += pcnt

    @pl.when(i == (_N // _B) - 1)
    def _():
        pooled = sums[...] / jnp.maximum(cnts[...], 1.0)
        mu = jnp.mean(pooled, axis=1, keepdims=True)
        var = jnp.mean((pooled - mu) ** 2, axis=1, keepdims=True)
        xn = (pooled - mu) / jnp.sqrt(var + 1e-5)
        o_ref[...] = (
            jnp.dot(xn, wl_ref[...], preferred_element_type=jnp.float32)
            + bl_ref[...]
        )


def _final(p0, p1, h, d0, d1, b, batchf, wl, bl2):
    din = h.shape[1]
    return pl.pallas_call(
        _final_body,
        grid=(_N // _B,),
        in_specs=[
            pl.BlockSpec((_B, din), lambda i: (i, 0)),
            pl.BlockSpec((_B, din), lambda i: (i, 0)),
            pl.BlockSpec((_B, din), lambda i: (i, 0)),
            pl.BlockSpec((_B, _DEG_W), lambda i: (i, 0)),
            pl.BlockSpec((_B, _DEG_W), lambda i: (i, 0)),
            pl.BlockSpec((1, din), lambda i: (0, 0)),
            pl.BlockSpec((_B, _DEG_W), lambda i: (i, 0)),
            pl.BlockSpec((din, 1), lambda i: (0, 0)),
            pl.BlockSpec((1, 1), lambda i: (0, 0)),
        ],
        out_specs=pl.BlockSpec((_G, 1), lambda i: (0, 0)),
        out_shape=jax.ShapeDtypeStruct((_G, 1), jnp.float32),
        scratch_shapes=[
            pltpu.VMEM((_G, din), jnp.float32),
            pltpu.VMEM((_G, din), jnp.float32),
        ],
    )(p0, p1, h, d0, d1, b, batchf, wl, bl2)


def kernel(x, edge_index, batch, W1, b1, W2, b2, W3, b3, W4, b4, W5, b5, Wl, bl):
    e = edge_index.shape[1]
    n_chunks = -(-e // (_NW * _CHUNK))
    n_chunks = -(-n_chunks // 4) * 4
    e_pad = n_chunks * _NW * _CHUNK
    pad = e_pad - e
    pad_dst = _N + (jnp.arange(pad, dtype=jnp.int32) % (_N_PAD - _N))
    pad_src = jnp.arange(pad, dtype=jnp.int32) % _N
    src = jnp.concatenate([edge_index[0], pad_src]).reshape(_NW, n_chunks, _CHUNK)
    dst = jnp.concatenate([edge_index[1], pad_dst]).reshape(_NW, n_chunks, _CHUNK)

    zeros128 = jnp.zeros((_CHUNK, 128), jnp.float32)
    onesd = jnp.ones((_CHUNK, _DEG_SC), jnp.float32)

    dpart = _make_deg(n_chunks)(dst, onesd, zeros128)
    d0 = dpart[0, :_N, :_DEG_W]
    d1 = dpart[1, :_N, :_DEG_W]

    def pad_w(w):
        return jnp.zeros((128, 128), jnp.float32).at[: w.shape[0], : w.shape[1]].set(w)

    def pad_b(b):
        return jnp.zeros((128,), jnp.float32).at[: b.shape[0]].set(b)

    scatter = _make_scatter(n_chunks, 128)
    h = _mm_first(x, d0, d1, W1)
    weights = [(b1, pad_w(W2)), (pad_b(b2), pad_w(W3)),
               (pad_b(b3), pad_w(W4)), (pad_b(b4), pad_w(W5))]
    for b_prev, w_next in weights:
        p = scatter(h, src, dst, zeros128)
        h = _layer(p[0, :_N], p[1, :_N], h, d0, d1, b_prev[None, :], w_next)

    p = scatter(h, src, dst, zeros128)
    batchf = jnp.broadcast_to(
        batch.astype(jnp.float32)[:, None], (_N, _DEG_W)
    )
    return _final(
        p[0, :_N], p[1, :_N], h, d0, d1, b5[None, :], batchf, Wl,
        bl.reshape(1, 1)
    )
